# Initial kernel scaffold; baseline (speedup 1.0000x reference)
import jax
import jax.numpy as jnp
from jax.experimental import pallas as pl


def kernel(input_nodes, edge_src0, edge_dst0, edge_src1, edge_dst1, emb, W0, b0, W1, b1, Wout, bout):
    raise NotImplementedError("write your pallas kernel here")


# speedup vs baseline: 3.2023x; 3.2023x over previous
import jax
import jax.numpy as jnp
from jax import lax
from jax.experimental import pallas as pl
from jax.experimental.pallas import tpu as pltpu
from jax.experimental.pallas import tpu_sc as plsc

_N0, _N1, _N2 = 10000, 4000, 1000
_E0, _E1 = 160000, 64000
_NUM_NODES, _HID, _OUT = 100000, 256, 4
_NC, _NS, _L = 2, 16, 16
_DH = _HID // _NC

_N0P = _NS * 5 * 128
_N1P = _NS * 256
_N2P = _NS * 64
_C0 = 80
_C1 = 32
_E0P = _NS * _C0 * 128
_E1P = _NS * _C1 * 128

_F32 = jnp.float32
_I32 = jnp.int32


def _rsqrt16(x):
    i = lax.bitcast_convert_type(x, _I32)
    y = lax.bitcast_convert_type(jnp.int32(0x5F3759DF) - (i >> 1), _F32)
    for _ in range(3):
        y = y * (1.5 - 0.5 * x * y * y)
    return y


def _zero_vmem_2d(buf, rows):
    zv = jnp.zeros((16,), _F32)

    @pl.loop(0, rows)
    def _(i):
        for k in range(8):
            buf[i, pl.ds(k * 16, 16)] = zv


def _zero_vmem_1d(buf, n):
    zv = jnp.zeros((16,), _F32)

    @pl.loop(0, n // 16)
    def _(i):
        buf[pl.ds(i * 16, 16)] = zv


def _sc1_body(emb2, inp, esrc0, edst0, esrc1,
              agg_out, din0_out, dout1_out, h0_out,
              esrc_b, edst_b, esrc1_b, inp_b, gbuf, scale_b, ones_b,
              agg_sp, hsrc_sp, hdst_sp, hout1_sp, gsem):
    c = lax.axis_index("c")
    s = lax.axis_index("s")

    pltpu.sync_copy(inp.at[s], inp_b)
    pltpu.sync_copy(esrc0.at[s], esrc_b)
    pltpu.sync_copy(edst0.at[s], edst_b)
    pltpu.sync_copy(esrc1.at[s], esrc1_b)

    _zero_vmem_2d(gbuf, 128)
    _zero_vmem_1d(scale_b, 640)

    @pl.loop(0, 8)
    def _(i):
        ones_b[pl.ds(i * 16, 16)] = jnp.ones((16,), _F32)

    pltpu.sync_copy(gbuf, agg_sp.at[pl.ds(s * 256, 128)])
    pltpu.sync_copy(gbuf, agg_sp.at[pl.ds(s * 256 + 128, 128)])
    pltpu.sync_copy(scale_b, hsrc_sp.at[pl.ds(s * 640, 640)])
    pltpu.sync_copy(scale_b.at[pl.ds(0, 256)], hdst_sp.at[pl.ds(s * 256, 256)])
    pltpu.sync_copy(scale_b.at[pl.ds(0, 256)], hout1_sp.at[pl.ds(s * 256, 256)])
    plsc.subcore_barrier()

    @pl.loop(0, _C0)
    def _(j):
        pltpu.sync_copy(ones_b, hsrc_sp.at[esrc_b.at[j]], add=True)
        pltpu.sync_copy(ones_b, hdst_sp.at[edst_b.at[j]], add=True)

    @pl.loop(0, _C1)
    def _(j):
        pltpu.sync_copy(ones_b, hout1_sp.at[esrc1_b.at[j]], add=True)

    plsc.subcore_barrier()

    pltpu.sync_copy(hsrc_sp.at[pl.ds(s * 640, 640)], scale_b)

    @pl.loop(0, 40)
    def _(i):
        d = scale_b[pl.ds(i * 16, 16)]
        scale_b[pl.ds(i * 16, 16)] = _rsqrt16(jnp.maximum(d, 1.0))

    for r in range(5):
        @pl.loop(0, 8)
        def _(k, _r=r):
            v = inp_b[_r, pl.ds(k * 16, 16)]
            inp_b[_r, pl.ds(k * 16, 16)] = v * 2 + c

        pltpu.async_copy(emb2.at[inp_b.at[r]], gbuf, gsem).wait()

        @pl.loop(0, 128)
        def _(i, _r=r):
            sc = plsc.load_gather(
                scale_b, [jnp.full((16,), _r * 128 + i, _I32)])
            for k in range(8):
                gbuf[i, pl.ds(k * 16, 16)] = gbuf[i, pl.ds(k * 16, 16)] * sc

        pltpu.sync_copy(gbuf,
                        h0_out.at[pl.ds(c * _N0P + s * 640 + r * 128, 128)])

    @pl.loop(0, _C0)
    def _(j):
        for k in range(8):
            v = esrc_b[j, pl.ds(k * 16, 16)]
            esrc_b[j, pl.ds(k * 16, 16)] = v + c * _N0P

    plsc.subcore_barrier()

    @pl.loop(0, _C0)
    def _(j):
        pltpu.async_copy(h0_out.at[esrc_b.at[j]], gbuf, gsem).wait()
        pltpu.sync_copy(gbuf, agg_sp.at[edst_b.at[j]], add=True)

    plsc.subcore_barrier()

    pltpu.sync_copy(agg_sp.at[pl.ds(s * 256, 256)],
                    agg_out.at[c, pl.ds(s * 256, 256)])

    @pl.when(jnp.logical_and(c == 0, s == 0))
    def _():
        pltpu.sync_copy(hdst_sp, din0_out)
        pltpu.sync_copy(hout1_sp, dout1_out)


def _sc2_body(h1p, esrc1, edst1,
              agg_out, din1_out,
              esrc_b, edst_b, gbuf, zb, ones_b,
              h1_sp, agg_sp, hdst_sp, gsem):
    c = lax.axis_index("c")
    s = lax.axis_index("s")

    pltpu.sync_copy(esrc1.at[s], esrc_b)
    pltpu.sync_copy(edst1.at[s], edst_b)

    pltpu.sync_copy(h1p.at[c, pl.ds(s * 256, 256)],
                    h1_sp.at[pl.ds(s * 256, 256)])

    _zero_vmem_2d(gbuf, 64)
    _zero_vmem_1d(zb, 64)

    @pl.loop(0, 8)
    def _(i):
        ones_b[pl.ds(i * 16, 16)] = jnp.ones((16,), _F32)

    pltpu.sync_copy(gbuf.at[pl.ds(0, 64)], agg_sp.at[pl.ds(s * 64, 64)])
    pltpu.sync_copy(zb, hdst_sp.at[pl.ds(s * 64, 64)])
    plsc.subcore_barrier()

    @pl.loop(0, _C1)
    def _(j):
        pltpu.sync_copy(ones_b, hdst_sp.at[edst_b.at[j]], add=True)

    plsc.subcore_barrier()

    @pl.loop(0, _C1)
    def _(j):
        pltpu.async_copy(h1_sp.at[esrc_b.at[j]], gbuf, gsem).wait()
        pltpu.sync_copy(gbuf, agg_sp.at[edst_b.at[j]], add=True)

    plsc.subcore_barrier()

    pltpu.sync_copy(agg_sp.at[pl.ds(s * 64, 64)],
                    agg_out.at[c, pl.ds(s * 64, 64)])

    @pl.when(jnp.logical_and(c == 0, s == 0))
    def _():
        pltpu.sync_copy(hdst_sp, din1_out)


_MESH = plsc.VectorSubcoreMesh(core_axis_name="c", subcore_axis_name="s",
                               num_cores=_NC, num_subcores=_NS)
_SC_PARAMS = pltpu.CompilerParams(needs_layout_passes=False)

_sc1 = pl.kernel(
    _sc1_body,
    out_type=[
        jax.ShapeDtypeStruct((_NC, _N1P, _DH), _F32),
        jax.ShapeDtypeStruct((_N1P,), _F32),
        jax.ShapeDtypeStruct((_N1P,), _F32),
        jax.ShapeDtypeStruct((_NC * _N0P, _DH), _F32),
    ],
    mesh=_MESH,
    compiler_params=_SC_PARAMS,
    scratch_types=[
        pltpu.VMEM((_C0, 128), _I32),
        pltpu.VMEM((_C0, 128), _I32),
        pltpu.VMEM((_C1, 128), _I32),
        pltpu.VMEM((5, 128), _I32),
        pltpu.VMEM((128, 128), _F32),
        pltpu.VMEM((640,), _F32),
        pltpu.VMEM((128,), _F32),
        pltpu.VMEM_SHARED((_N1P, _DH), _F32),
        pltpu.VMEM_SHARED((_N0P,), _F32),
        pltpu.VMEM_SHARED((_N1P,), _F32),
        pltpu.VMEM_SHARED((_N1P,), _F32),
        pltpu.SemaphoreType.DMA,
    ],
)

_sc2 = pl.kernel(
    _sc2_body,
    out_type=[
        jax.ShapeDtypeStruct((_NC, _N2P, _DH), _F32),
        jax.ShapeDtypeStruct((_N2P,), _F32),
    ],
    mesh=_MESH,
    compiler_params=_SC_PARAMS,
    scratch_types=[
        pltpu.VMEM((_C1, 128), _I32),
        pltpu.VMEM((_C1, 128), _I32),
        pltpu.VMEM((128, 128), _F32),
        pltpu.VMEM((64,), _F32),
        pltpu.VMEM((128,), _F32),
        pltpu.VMEM_SHARED((_N1P, _DH), _F32),
        pltpu.VMEM_SHARED((_N2P, _DH), _F32),
        pltpu.VMEM_SHARED((_N2P,), _F32),
        pltpu.SemaphoreType.DMA,
    ],
)


def _tc1_body(agg_ref, din_ref, dout_ref, w_ref, b_ref, out_ref):
    x = jnp.concatenate([agg_ref[0], agg_ref[1]], axis=-1)
    x = x * lax.rsqrt(jnp.maximum(din_ref[...], 1.0))
    y = jnp.dot(x, w_ref[...], preferred_element_type=_F32) + b_ref[...]
    y = jnp.maximum(y, 0.0) * lax.rsqrt(jnp.maximum(dout_ref[...], 1.0))
    out_ref[0] = y[:, :_DH]
    out_ref[1] = y[:, _DH:]


def _tc2_body(agg_ref, din_ref, w1_ref, b1_ref, wo_ref, bo_ref, out_ref):
    x = jnp.concatenate([agg_ref[0], agg_ref[1]], axis=-1)
    x = x * lax.rsqrt(jnp.maximum(din_ref[...], 1.0))
    h = jnp.dot(x, w1_ref[...], preferred_element_type=_F32) + b1_ref[...]
    y = jnp.dot(h, wo_ref[...], preferred_element_type=_F32) + bo_ref[...]
    out_ref[...] = y[:_N2]


_tc1 = pl.pallas_call(
    _tc1_body,
    out_shape=jax.ShapeDtypeStruct((_NC, _N1P, _DH), _F32),
)

_tc2 = pl.pallas_call(
    _tc2_body,
    out_shape=jax.ShapeDtypeStruct((_N2, _OUT), _F32),
)


def kernel(input_nodes, edge_src0, edge_dst0, edge_src1, edge_dst1,
           emb, W0, b0, W1, b1, Wout, bout):
    emb2 = emb.reshape(_NUM_NODES * _NC, _DH)
    inp = jnp.concatenate(
        [input_nodes.astype(_I32), jnp.zeros((_N0P - _N0,), _I32)]
    ).reshape(_NS, 5, 128)
    es0 = jnp.concatenate(
        [edge_src0.astype(_I32), jnp.full((_E0P - _E0,), _N0, _I32)]
    ).reshape(_NS, _C0, 128)
    ed0 = jnp.concatenate(
        [edge_dst0.astype(_I32), jnp.full((_E0P - _E0,), _N1, _I32)]
    ).reshape(_NS, _C0, 128)
    es1 = jnp.concatenate(
        [edge_src1.astype(_I32), jnp.full((_E1P - _E1,), _N1, _I32)]
    ).reshape(_NS, _C1, 128)
    ed1 = jnp.concatenate(
        [edge_dst1.astype(_I32), jnp.full((_E1P - _E1,), _N2, _I32)]
    ).reshape(_NS, _C1, 128)

    agg0, din0, dout1, _ = _sc1(emb2, inp, es0, ed0, es1)
    h1 = _tc1(agg0, din0.reshape(_N1P, 1), dout1.reshape(_N1P, 1),
              W0, b0.reshape(1, _HID))
    agg1, din1 = _sc2(h1, es1, ed1)
    logits = _tc2(agg1, din1.reshape(_N2P, 1),
                  W1, b1.reshape(1, _HID), Wout, bout.reshape(1, _OUT))
    return logits


# speedup vs baseline: 3.5509x; 1.1089x over previous
import jax
import jax.numpy as jnp
from jax import lax
from jax.experimental import pallas as pl
from jax.experimental.pallas import tpu as pltpu
from jax.experimental.pallas import tpu_sc as plsc

_N0, _N1, _N2 = 10000, 4000, 1000
_E0, _E1 = 160000, 64000
_NUM_NODES, _HID, _OUT = 100000, 256, 4
_NC, _NS, _L = 2, 16, 16
_DH = _HID // _NC

_N0P = _NS * 5 * 128
_N1P = _NS * 256
_N2P = _NS * 64
_C0 = 80
_C1 = 32
_E0P = _NS * _C0 * 128
_E1P = _NS * _C1 * 128

_F32 = jnp.float32
_I32 = jnp.int32


def _rsqrt16(x):
    i = lax.bitcast_convert_type(x, _I32)
    y = lax.bitcast_convert_type(jnp.int32(0x5F3759DF) - (i >> 1), _F32)
    for _ in range(3):
        y = y * (1.5 - 0.5 * x * y * y)
    return y


def _zero_vmem_2d(buf, rows):
    zv = jnp.zeros((16,), _F32)

    @pl.loop(0, rows)
    def _(i):
        for k in range(8):
            buf[i, pl.ds(k * 16, 16)] = zv


def _zero_vmem_1d(buf, n):
    zv = jnp.zeros((16,), _F32)

    @pl.loop(0, n // 16)
    def _(i):
        buf[pl.ds(i * 16, 16)] = zv


def _hist_chunks(idx_b, n_chunks, hist_sp, ones_b, sem):
    assert n_chunks % 8 == 0

    @pl.loop(0, n_chunks // 8)
    def _(i):
        descs = [
            pltpu.async_copy(ones_b, hist_sp.at[idx_b.at[i * 8 + k]], sem,
                             add=True)
            for k in range(8)
        ]
        for d in descs:
            d.wait()


def _edge_pipeline(n_chunks, src_hbm, esrc_b, edst_b, agg_sp,
                   gbuf0, gbuf1, gsem0, gsem1, ssem0, ssem1):
    assert n_chunks % 2 == 0
    half = n_chunks // 2
    pltpu.async_copy(src_hbm.at[esrc_b.at[0]], gbuf0, gsem0)
    pltpu.async_copy(src_hbm.at[esrc_b.at[1]], gbuf1, gsem1)

    @pl.loop(0, half)
    def _(i):
        j0 = 2 * i
        j1 = 2 * i + 1
        pltpu.make_async_copy(src_hbm.at[esrc_b.at[j0]], gbuf0, gsem0).wait()
        d0 = pltpu.async_copy(gbuf0, agg_sp.at[edst_b.at[j0]], ssem0,
                              add=True)
        pltpu.make_async_copy(src_hbm.at[esrc_b.at[j1]], gbuf1, gsem1).wait()
        d1 = pltpu.async_copy(gbuf1, agg_sp.at[edst_b.at[j1]], ssem1,
                              add=True)
        d0.wait()

        @pl.when(i < half - 1)
        def _():
            pltpu.async_copy(src_hbm.at[esrc_b.at[j0 + 2]], gbuf0, gsem0)

        d1.wait()

        @pl.when(i < half - 1)
        def _():
            pltpu.async_copy(src_hbm.at[esrc_b.at[j1 + 2]], gbuf1, gsem1)


def _sc1_body(emb2, inp, esrc0, edst0, esrc1,
              agg_out, din0_out, dout1_out, h0_out,
              esrc_b, edst_b, esrc1_b, inp_b, gbuf0, gbuf1, scale_b, ones_b,
              agg_sp, hsrc_sp, hdst_sp, hout1_sp,
              gsem0, gsem1, ssem0, ssem1, hsem):
    c = lax.axis_index("c")
    s = lax.axis_index("s")

    pltpu.sync_copy(inp.at[s], inp_b)
    pltpu.sync_copy(esrc0.at[s], esrc_b)
    pltpu.sync_copy(edst0.at[s], edst_b)
    pltpu.sync_copy(esrc1.at[s], esrc1_b)

    _zero_vmem_2d(gbuf0, 128)
    _zero_vmem_1d(scale_b, 640)

    @pl.loop(0, 8)
    def _(i):
        ones_b[pl.ds(i * 16, 16)] = jnp.ones((16,), _F32)

    pltpu.sync_copy(gbuf0, agg_sp.at[pl.ds(s * 256, 128)])
    pltpu.sync_copy(gbuf0, agg_sp.at[pl.ds(s * 256 + 128, 128)])
    pltpu.sync_copy(scale_b, hsrc_sp.at[pl.ds(s * 640, 640)])
    pltpu.sync_copy(scale_b.at[pl.ds(0, 256)], hdst_sp.at[pl.ds(s * 256, 256)])
    pltpu.sync_copy(scale_b.at[pl.ds(0, 256)], hout1_sp.at[pl.ds(s * 256, 256)])
    plsc.subcore_barrier()

    _hist_chunks(esrc_b, _C0, hsrc_sp, ones_b, hsem)
    _hist_chunks(edst_b, _C0, hdst_sp, ones_b, hsem)
    _hist_chunks(esrc1_b, _C1, hout1_sp, ones_b, hsem)
    plsc.subcore_barrier()

    pltpu.sync_copy(hsrc_sp.at[pl.ds(s * 640, 640)], scale_b)

    @pl.loop(0, 40)
    def _(i):
        d = scale_b[pl.ds(i * 16, 16)]
        scale_b[pl.ds(i * 16, 16)] = _rsqrt16(jnp.maximum(d, 1.0))

    @pl.loop(0, 5)
    def _(r):
        for k in range(8):
            v = inp_b[r, pl.ds(k * 16, 16)]
            inp_b[r, pl.ds(k * 16, 16)] = v * 2 + c

    bufs = (gbuf0, gbuf1)
    sems = (gsem0, gsem1)
    pltpu.async_copy(emb2.at[inp_b.at[0]], gbuf0, gsem0)
    for r in range(5):
        b, g = bufs[r % 2], sems[r % 2]
        pltpu.make_async_copy(emb2.at[inp_b.at[r]], b, g).wait()
        if r + 1 < 5:
            pltpu.async_copy(emb2.at[inp_b.at[r + 1]], bufs[(r + 1) % 2],
                             sems[(r + 1) % 2])

        @pl.loop(0, 128)
        def _(i, _r=r, _b=b):
            sc = plsc.load_gather(
                scale_b, [jnp.full((16,), _r * 128 + i, _I32)])
            for k in range(8):
                _b[i, pl.ds(k * 16, 16)] = _b[i, pl.ds(k * 16, 16)] * sc

        pltpu.async_copy(
            b, h0_out.at[pl.ds(c * _N0P + s * 640 + r * 128, 128)],
            ssem0).wait()

    @pl.loop(0, _C0)
    def _(j):
        for k in range(8):
            v = esrc_b[j, pl.ds(k * 16, 16)]
            esrc_b[j, pl.ds(k * 16, 16)] = v + c * _N0P

    plsc.subcore_barrier()

    _edge_pipeline(_C0, h0_out, esrc_b, edst_b, agg_sp,
                   gbuf0, gbuf1, gsem0, gsem1, ssem0, ssem1)
    plsc.subcore_barrier()

    pltpu.sync_copy(agg_sp.at[pl.ds(s * 256, 256)],
                    agg_out.at[c, pl.ds(s * 256, 256)])

    @pl.when(jnp.logical_and(c == 0, s == 0))
    def _():
        pltpu.sync_copy(hdst_sp, din0_out)
        pltpu.sync_copy(hout1_sp, dout1_out)


def _sc2_body(h1p, esrc1, edst1,
              agg_out, din1_out,
              esrc_b, edst_b, gbuf0, gbuf1, zb, ones_b,
              h1_sp, agg_sp, hdst_sp,
              gsem0, gsem1, ssem0, ssem1, hsem):
    c = lax.axis_index("c")
    s = lax.axis_index("s")

    pltpu.sync_copy(esrc1.at[s], esrc_b)
    pltpu.sync_copy(edst1.at[s], edst_b)

    pltpu.async_copy(h1p.at[c, pl.ds(s * 256, 256)],
                     h1_sp.at[pl.ds(s * 256, 256)], gsem0)

    _zero_vmem_2d(gbuf0, 64)
    _zero_vmem_1d(zb, 64)

    @pl.loop(0, 8)
    def _(i):
        ones_b[pl.ds(i * 16, 16)] = jnp.ones((16,), _F32)

    pltpu.sync_copy(gbuf0.at[pl.ds(0, 64)], agg_sp.at[pl.ds(s * 64, 64)])
    pltpu.sync_copy(zb, hdst_sp.at[pl.ds(s * 64, 64)])
    pltpu.make_async_copy(h1p.at[c, pl.ds(s * 256, 256)],
                          h1_sp.at[pl.ds(s * 256, 256)], gsem0).wait()
    plsc.subcore_barrier()

    _hist_chunks(edst_b, _C1, hdst_sp, ones_b, hsem)
    plsc.subcore_barrier()

    _edge_pipeline(_C1, h1_sp, esrc_b, edst_b, agg_sp,
                   gbuf0, gbuf1, gsem0, gsem1, ssem0, ssem1)
    plsc.subcore_barrier()

    pltpu.sync_copy(agg_sp.at[pl.ds(s * 64, 64)],
                    agg_out.at[c, pl.ds(s * 64, 64)])

    @pl.when(jnp.logical_and(c == 0, s == 0))
    def _():
        pltpu.sync_copy(hdst_sp, din1_out)


_MESH = plsc.VectorSubcoreMesh(core_axis_name="c", subcore_axis_name="s",
                               num_cores=_NC, num_subcores=_NS)
_SC_PARAMS = pltpu.CompilerParams(needs_layout_passes=False)

_sc1 = pl.kernel(
    _sc1_body,
    out_type=[
        jax.ShapeDtypeStruct((_NC, _N1P, _DH), _F32),
        jax.ShapeDtypeStruct((_N1P,), _F32),
        jax.ShapeDtypeStruct((_N1P,), _F32),
        jax.ShapeDtypeStruct((_NC * _N0P, _DH), _F32),
    ],
    mesh=_MESH,
    compiler_params=_SC_PARAMS,
    scratch_types=[
        pltpu.VMEM((_C0, 128), _I32),
        pltpu.VMEM((_C0, 128), _I32),
        pltpu.VMEM((_C1, 128), _I32),
        pltpu.VMEM((5, 128), _I32),
        pltpu.VMEM((128, 128), _F32),
        pltpu.VMEM((128, 128), _F32),
        pltpu.VMEM((640,), _F32),
        pltpu.VMEM((128,), _F32),
        pltpu.VMEM_SHARED((_N1P, _DH), _F32),
        pltpu.VMEM_SHARED((_N0P,), _F32),
        pltpu.VMEM_SHARED((_N1P,), _F32),
        pltpu.VMEM_SHARED((_N1P,), _F32),
        pltpu.SemaphoreType.DMA,
        pltpu.SemaphoreType.DMA,
        pltpu.SemaphoreType.DMA,
        pltpu.SemaphoreType.DMA,
        pltpu.SemaphoreType.DMA,
    ],
)

_sc2 = pl.kernel(
    _sc2_body,
    out_type=[
        jax.ShapeDtypeStruct((_NC, _N2P, _DH), _F32),
        jax.ShapeDtypeStruct((_N2P,), _F32),
    ],
    mesh=_MESH,
    compiler_params=_SC_PARAMS,
    scratch_types=[
        pltpu.VMEM((_C1, 128), _I32),
        pltpu.VMEM((_C1, 128), _I32),
        pltpu.VMEM((128, 128), _F32),
        pltpu.VMEM((128, 128), _F32),
        pltpu.VMEM((64,), _F32),
        pltpu.VMEM((128,), _F32),
        pltpu.VMEM_SHARED((_N1P, _DH), _F32),
        pltpu.VMEM_SHARED((_N2P, _DH), _F32),
        pltpu.VMEM_SHARED((_N2P,), _F32),
        pltpu.SemaphoreType.DMA,
        pltpu.SemaphoreType.DMA,
        pltpu.SemaphoreType.DMA,
        pltpu.SemaphoreType.DMA,
        pltpu.SemaphoreType.DMA,
    ],
)


def _tc1_body(agg_ref, din_ref, dout_ref, w_ref, b_ref, out_ref):
    x = jnp.concatenate([agg_ref[0], agg_ref[1]], axis=-1)
    x = x * lax.rsqrt(jnp.maximum(din_ref[...], 1.0))
    y = jnp.dot(x, w_ref[...], preferred_element_type=_F32) + b_ref[...]
    y = jnp.maximum(y, 0.0) * lax.rsqrt(jnp.maximum(dout_ref[...], 1.0))
    out_ref[0] = y[:, :_DH]
    out_ref[1] = y[:, _DH:]


def _tc2_body(agg_ref, din_ref, w1_ref, b1_ref, wo_ref, bo_ref, out_ref):
    x = jnp.concatenate([agg_ref[0], agg_ref[1]], axis=-1)
    x = x * lax.rsqrt(jnp.maximum(din_ref[...], 1.0))
    h = jnp.dot(x, w1_ref[...], preferred_element_type=_F32) + b1_ref[...]
    y = jnp.dot(h, wo_ref[...], preferred_element_type=_F32) + bo_ref[...]
    out_ref[...] = y[:_N2]


_tc1 = pl.pallas_call(
    _tc1_body,
    out_shape=jax.ShapeDtypeStruct((_NC, _N1P, _DH), _F32),
)

_tc2 = pl.pallas_call(
    _tc2_body,
    out_shape=jax.ShapeDtypeStruct((_N2, _OUT), _F32),
)


def kernel(input_nodes, edge_src0, edge_dst0, edge_src1, edge_dst1,
           emb, W0, b0, W1, b1, Wout, bout):
    emb2 = emb.reshape(_NUM_NODES * _NC, _DH)
    inp = jnp.concatenate(
        [input_nodes.astype(_I32), jnp.zeros((_N0P - _N0,), _I32)]
    ).reshape(_NS, 5, 128)
    es0 = jnp.concatenate(
        [edge_src0.astype(_I32), jnp.full((_E0P - _E0,), _N0, _I32)]
    ).reshape(_NS, _C0, 128)
    ed0 = jnp.concatenate(
        [edge_dst0.astype(_I32), jnp.full((_E0P - _E0,), _N1, _I32)]
    ).reshape(_NS, _C0, 128)
    es1 = jnp.concatenate(
        [edge_src1.astype(_I32), jnp.full((_E1P - _E1,), _N1, _I32)]
    ).reshape(_NS, _C1, 128)
    ed1 = jnp.concatenate(
        [edge_dst1.astype(_I32), jnp.full((_E1P - _E1,), _N2, _I32)]
    ).reshape(_NS, _C1, 128)

    agg0, din0, dout1, _ = _sc1(emb2, inp, es0, ed0, es1)
    h1 = _tc1(agg0, din0.reshape(_N1P, 1), dout1.reshape(_N1P, 1),
              W0, b0.reshape(1, _HID))
    agg1, din1 = _sc2(h1, es1, ed1)
    logits = _tc2(agg1, din1.reshape(_N2P, 1),
                  W1, b1.reshape(1, _HID), Wout, bout.reshape(1, _OUT))
    return logits


# speedup vs baseline: 3.7209x; 1.0479x over previous
import jax
import jax.numpy as jnp
from jax import lax
from jax.experimental import pallas as pl
from jax.experimental.pallas import tpu as pltpu
from jax.experimental.pallas import tpu_sc as plsc

_N0, _N1, _N2 = 10000, 4000, 1000
_E0, _E1 = 160000, 64000
_NUM_NODES, _HID, _OUT = 100000, 256, 4
_NC, _NS, _L = 2, 16, 16
_DH = _HID // _NC

_N0P = _NS * 5 * 128
_N1P = _NS * 256
_N2P = _NS * 64
_C0 = 80
_C1 = 32
_E0P = _NS * _C0 * 128
_E1P = _NS * _C1 * 128

_F32 = jnp.float32
_I32 = jnp.int32


def _rsqrt16(x):
    i = lax.bitcast_convert_type(x, _I32)
    y = lax.bitcast_convert_type(jnp.int32(0x5F3759DF) - (i >> 1), _F32)
    for _ in range(3):
        y = y * (1.5 - 0.5 * x * y * y)
    return y


def _zero_vmem_2d(buf, rows):
    zv = jnp.zeros((16,), _F32)

    @pl.loop(0, rows)
    def _(i):
        for k in range(8):
            buf[i, pl.ds(k * 16, 16)] = zv


def _zero_vmem_1d(buf, n):
    zv = jnp.zeros((16,), _F32)

    @pl.loop(0, n // 16)
    def _(i):
        buf[pl.ds(i * 16, 16)] = zv


def _hist_chunks(idx_b, n_chunks, hist_sp, ones_b, sem):
    assert n_chunks % 8 == 0

    @pl.loop(0, n_chunks // 8)
    def _(i):
        descs = [
            pltpu.async_copy(ones_b, hist_sp.at[idx_b.at[i * 8 + k]], sem,
                             add=True)
            for k in range(8)
        ]
        for d in descs:
            d.wait()


def _edge_pipeline4(n_chunks, src, esrc_b, edst_b, agg_sp,
                    gbs, gsems, ssems, extra=None):
    assert n_chunks % 4 == 0
    nq = n_chunks // 4
    for k in range(4):
        pltpu.async_copy(src.at[esrc_b.at[k]], gbs[k], gsems[k])

    @pl.loop(0, nq)
    def _(i):
        base = i * 4
        ds = []
        for k in range(4):
            pltpu.make_async_copy(src.at[esrc_b.at[base + k]],
                                  gbs[k], gsems[k]).wait()
            ds.append(pltpu.async_copy(gbs[k], agg_sp.at[edst_b.at[base + k]],
                                       ssems[k], add=True))
        if extra is not None:
            extra(i)
        for k in range(4):
            ds[k].wait()

            @pl.when(base + k + 4 < n_chunks)
            def _(_k=k):
                pltpu.async_copy(src.at[esrc_b.at[base + 4 + _k]],
                                 gbs[_k], gsems[_k])


def _sc1_body(emb2, inp, esrc0, edst0, esrc1,
              agg_out, din0_out, dout1_out, h0_out,
              esrc_b, edst_b, esrc1_b, inp_b,
              gb0, gb1, gb2, gb3, scale_b, ones_b,
              agg_sp, hsrc_sp, hdst_sp, hout1_sp,
              gsem0, gsem1, gsem2, gsem3,
              ssem0, ssem1, ssem2, ssem3, wsem, hsem, stsem):
    c = lax.axis_index("c")
    s = lax.axis_index("s")
    gbs = (gb0, gb1, gb2, gb3)
    gsems = (gsem0, gsem1, gsem2, gsem3)
    ssems = (ssem0, ssem1, ssem2, ssem3)

    pltpu.sync_copy(esrc0.at[s], esrc_b)
    d_ed = pltpu.async_copy(edst0.at[s], edst_b, stsem)
    d_e1 = pltpu.async_copy(esrc1.at[s], esrc1_b, stsem)
    d_in = pltpu.async_copy(inp.at[s], inp_b, stsem)

    _zero_vmem_2d(gb0, 128)
    _zero_vmem_1d(scale_b, 640)

    @pl.loop(0, 8)
    def _(i):
        ones_b[pl.ds(i * 16, 16)] = jnp.ones((16,), _F32)

    pltpu.sync_copy(gb0, agg_sp.at[pl.ds(s * 256, 128)])
    pltpu.sync_copy(gb0, agg_sp.at[pl.ds(s * 256 + 128, 128)])
    pltpu.sync_copy(scale_b, hsrc_sp.at[pl.ds(s * 640, 640)])
    pltpu.sync_copy(scale_b.at[pl.ds(0, 256)], hdst_sp.at[pl.ds(s * 256, 256)])
    pltpu.sync_copy(scale_b.at[pl.ds(0, 256)], hout1_sp.at[pl.ds(s * 256, 256)])

    d_ed.wait()
    d_e1.wait()
    d_in.wait()

    @pl.loop(0, 5)
    def _(r):
        for k in range(8):
            v = inp_b[r, pl.ds(k * 16, 16)]
            inp_b[r, pl.ds(k * 16, 16)] = v * 2 + c

    plsc.subcore_barrier()

    for r in range(4):
        pltpu.async_copy(emb2.at[inp_b.at[r]], gbs[r], gsems[r])

    _hist_chunks(esrc_b, _C0, hsrc_sp, ones_b, hsem)
    plsc.subcore_barrier()

    pltpu.sync_copy(hsrc_sp.at[pl.ds(s * 640, 640)], scale_b)

    @pl.loop(0, 40)
    def _(i):
        d = scale_b[pl.ds(i * 16, 16)]
        scale_b[pl.ds(i * 16, 16)] = _rsqrt16(jnp.maximum(d, 1.0))

    wdescs = []
    for r in range(5):
        b, g = gbs[r % 4], gsems[r % 4]
        pltpu.make_async_copy(emb2.at[inp_b.at[r]], b, g).wait()

        @pl.loop(0, 128)
        def _(i, _r=r, _b=b):
            sc = plsc.load_gather(
                scale_b, [jnp.full((16,), _r * 128 + i, _I32)])
            for k in range(8):
                _b[i, pl.ds(k * 16, 16)] = _b[i, pl.ds(k * 16, 16)] * sc

        wdescs.append(pltpu.async_copy(
            b, h0_out.at[pl.ds(c * _N0P + s * 640 + r * 128, 128)], wsem))
        if r == 0:
            wdescs[0].wait()
            pltpu.async_copy(emb2.at[inp_b.at[4]], gb0, gsem0)

    @pl.loop(0, _C0)
    def _(j):
        for k in range(8):
            v = esrc_b[j, pl.ds(k * 16, 16)]
            esrc_b[j, pl.ds(k * 16, 16)] = v + c * _N0P

    for d in wdescs[1:]:
        d.wait()
    plsc.subcore_barrier()

    def _extra(i):
        dh = [
            pltpu.async_copy(ones_b, hdst_sp.at[edst_b.at[i * 4 + k]], hsem,
                             add=True)
            for k in range(4)
        ]

        @pl.when(i < _C1 // 4)
        def _():
            d1 = [
                pltpu.async_copy(ones_b, hout1_sp.at[esrc1_b.at[i * 4 + k]],
                                 hsem, add=True)
                for k in range(4)
            ]
            for d in d1:
                d.wait()

        for d in dh:
            d.wait()

    _edge_pipeline4(_C0, h0_out, esrc_b, edst_b, agg_sp,
                    gbs, gsems, ssems, extra=_extra)
    plsc.subcore_barrier()

    pltpu.sync_copy(agg_sp.at[pl.ds(s * 256, 256)],
                    agg_out.at[c, pl.ds(s * 256, 256)])

    @pl.when(jnp.logical_and(c == 0, s == 0))
    def _():
        pltpu.sync_copy(hdst_sp, din0_out)
        pltpu.sync_copy(hout1_sp, dout1_out)


def _sc2_body(h1p, esrc1, edst1,
              agg_out, din1_out,
              esrc_b, edst_b, gb0, gb1, gb2, gb3, zb, ones_b,
              h1_sp, agg_sp, hdst_sp,
              gsem0, gsem1, gsem2, gsem3,
              ssem0, ssem1, ssem2, ssem3, hsem):
    c = lax.axis_index("c")
    s = lax.axis_index("s")
    gbs = (gb0, gb1, gb2, gb3)
    gsems = (gsem0, gsem1, gsem2, gsem3)
    ssems = (ssem0, ssem1, ssem2, ssem3)

    pltpu.sync_copy(esrc1.at[s], esrc_b)
    pltpu.sync_copy(edst1.at[s], edst_b)

    d_h1 = pltpu.async_copy(h1p.at[c, pl.ds(s * 256, 256)],
                            h1_sp.at[pl.ds(s * 256, 256)], gsem0)

    _zero_vmem_2d(gb1, 64)
    _zero_vmem_1d(zb, 64)

    @pl.loop(0, 8)
    def _(i):
        ones_b[pl.ds(i * 16, 16)] = jnp.ones((16,), _F32)

    pltpu.sync_copy(gb1.at[pl.ds(0, 64)], agg_sp.at[pl.ds(s * 64, 64)])
    pltpu.sync_copy(zb, hdst_sp.at[pl.ds(s * 64, 64)])
    plsc.subcore_barrier()

    _hist_chunks(edst_b, _C1, hdst_sp, ones_b, hsem)
    d_h1.wait()
    plsc.subcore_barrier()

    _edge_pipeline4(_C1, h1_sp, esrc_b, edst_b, agg_sp,
                    gbs, gsems, ssems)
    plsc.subcore_barrier()

    pltpu.sync_copy(agg_sp.at[pl.ds(s * 64, 64)],
                    agg_out.at[c, pl.ds(s * 64, 64)])

    @pl.when(jnp.logical_and(c == 0, s == 0))
    def _():
        pltpu.sync_copy(hdst_sp, din1_out)


_MESH = plsc.VectorSubcoreMesh(core_axis_name="c", subcore_axis_name="s",
                               num_cores=_NC, num_subcores=_NS)
_SC_PARAMS = pltpu.CompilerParams(needs_layout_passes=False)

_sc1 = pl.kernel(
    _sc1_body,
    out_type=[
        jax.ShapeDtypeStruct((_NC, _N1P, _DH), _F32),
        jax.ShapeDtypeStruct((_N1P,), _F32),
        jax.ShapeDtypeStruct((_N1P,), _F32),
        jax.ShapeDtypeStruct((_NC * _N0P, _DH), _F32),
    ],
    mesh=_MESH,
    compiler_params=_SC_PARAMS,
    scratch_types=[
        pltpu.VMEM((_C0, 128), _I32),
        pltpu.VMEM((_C0, 128), _I32),
        pltpu.VMEM((_C1, 128), _I32),
        pltpu.VMEM((5, 128), _I32),
        pltpu.VMEM((128, 128), _F32),
        pltpu.VMEM((128, 128), _F32),
        pltpu.VMEM((128, 128), _F32),
        pltpu.VMEM((128, 128), _F32),
        pltpu.VMEM((640,), _F32),
        pltpu.VMEM((128,), _F32),
        pltpu.VMEM_SHARED((_N1P, _DH), _F32),
        pltpu.VMEM_SHARED((_N0P,), _F32),
        pltpu.VMEM_SHARED((_N1P,), _F32),
        pltpu.VMEM_SHARED((_N1P,), _F32),
        pltpu.SemaphoreType.DMA,
        pltpu.SemaphoreType.DMA,
        pltpu.SemaphoreType.DMA,
        pltpu.SemaphoreType.DMA,
        pltpu.SemaphoreType.DMA,
        pltpu.SemaphoreType.DMA,
        pltpu.SemaphoreType.DMA,
        pltpu.SemaphoreType.DMA,
        pltpu.SemaphoreType.DMA,
        pltpu.SemaphoreType.DMA,
        pltpu.SemaphoreType.DMA,
    ],
)

_sc2 = pl.kernel(
    _sc2_body,
    out_type=[
        jax.ShapeDtypeStruct((_NC, _N2P, _DH), _F32),
        jax.ShapeDtypeStruct((_N2P,), _F32),
    ],
    mesh=_MESH,
    compiler_params=_SC_PARAMS,
    scratch_types=[
        pltpu.VMEM((_C1, 128), _I32),
        pltpu.VMEM((_C1, 128), _I32),
        pltpu.VMEM((128, 128), _F32),
        pltpu.VMEM((128, 128), _F32),
        pltpu.VMEM((128, 128), _F32),
        pltpu.VMEM((128, 128), _F32),
        pltpu.VMEM((64,), _F32),
        pltpu.VMEM((128,), _F32),
        pltpu.VMEM_SHARED((_N1P, _DH), _F32),
        pltpu.VMEM_SHARED((_N2P, _DH), _F32),
        pltpu.VMEM_SHARED((_N2P,), _F32),
        pltpu.SemaphoreType.DMA,
        pltpu.SemaphoreType.DMA,
        pltpu.SemaphoreType.DMA,
        pltpu.SemaphoreType.DMA,
        pltpu.SemaphoreType.DMA,
        pltpu.SemaphoreType.DMA,
        pltpu.SemaphoreType.DMA,
        pltpu.SemaphoreType.DMA,
        pltpu.SemaphoreType.DMA,
    ],
)


def _tc1_body(agg_ref, din_ref, dout_ref, w_ref, b_ref, out_ref):
    x = jnp.concatenate([agg_ref[0], agg_ref[1]], axis=-1)
    x = x * lax.rsqrt(jnp.maximum(din_ref[...], 1.0))
    y = jnp.dot(x, w_ref[...], preferred_element_type=_F32) + b_ref[...]
    y = jnp.maximum(y, 0.0) * lax.rsqrt(jnp.maximum(dout_ref[...], 1.0))
    out_ref[0] = y[:, :_DH]
    out_ref[1] = y[:, _DH:]


def _tc2_body(agg_ref, din_ref, w1_ref, b1_ref, wo_ref, bo_ref, out_ref):
    x = jnp.concatenate([agg_ref[0], agg_ref[1]], axis=-1)
    x = x * lax.rsqrt(jnp.maximum(din_ref[...], 1.0))
    h = jnp.dot(x, w1_ref[...], preferred_element_type=_F32) + b1_ref[...]
    y = jnp.dot(h, wo_ref[...], preferred_element_type=_F32) + bo_ref[...]
    out_ref[...] = y[:_N2]


_tc1 = pl.pallas_call(
    _tc1_body,
    out_shape=jax.ShapeDtypeStruct((_NC, _N1P, _DH), _F32),
)

_tc2 = pl.pallas_call(
    _tc2_body,
    out_shape=jax.ShapeDtypeStruct((_N2, _OUT), _F32),
)


def kernel(input_nodes, edge_src0, edge_dst0, edge_src1, edge_dst1,
           emb, W0, b0, W1, b1, Wout, bout):
    emb2 = emb.reshape(_NUM_NODES * _NC, _DH)
    inp = jnp.concatenate(
        [input_nodes.astype(_I32), jnp.zeros((_N0P - _N0,), _I32)]
    ).reshape(_NS, 5, 128)
    es0 = jnp.concatenate(
        [edge_src0.astype(_I32), jnp.full((_E0P - _E0,), _N0, _I32)]
    ).reshape(_NS, _C0, 128)
    ed0 = jnp.concatenate(
        [edge_dst0.astype(_I32), jnp.full((_E0P - _E0,), _N1, _I32)]
    ).reshape(_NS, _C0, 128)
    es1 = jnp.concatenate(
        [edge_src1.astype(_I32), jnp.full((_E1P - _E1,), _N1, _I32)]
    ).reshape(_NS, _C1, 128)
    ed1 = jnp.concatenate(
        [edge_dst1.astype(_I32), jnp.full((_E1P - _E1,), _N2, _I32)]
    ).reshape(_NS, _C1, 128)

    agg0, din0, dout1, _ = _sc1(emb2, inp, es0, ed0, es1)
    h1 = _tc1(agg0, din0.reshape(_N1P, 1), dout1.reshape(_N1P, 1),
              W0, b0.reshape(1, _HID))
    agg1, din1 = _sc2(h1, es1, ed1)
    logits = _tc2(agg1, din1.reshape(_N2P, 1),
                  W1, b1.reshape(1, _HID), Wout, bout.reshape(1, _OUT))
    return logits


# speedup vs baseline: 3.7372x; 1.0044x over previous
import jax
import jax.numpy as jnp
from jax import lax
from jax.experimental import pallas as pl
from jax.experimental.pallas import tpu as pltpu
from jax.experimental.pallas import tpu_sc as plsc

_N0, _N1, _N2 = 10000, 4000, 1000
_E0, _E1 = 160000, 64000
_NUM_NODES, _HID, _OUT = 100000, 256, 4
_NC, _NS, _L = 2, 16, 16
_DH = _HID // _NC

_N0P = _NS * 5 * 128
_N1P = _NS * 256
_N2P = _NS * 64
_C0 = 80
_C1 = 32
_E0P = _NS * _C0 * 128
_E1P = _NS * _C1 * 128

_F32 = jnp.float32
_I32 = jnp.int32


def _rsqrt16(x):
    i = lax.bitcast_convert_type(x, _I32)
    y = lax.bitcast_convert_type(jnp.int32(0x5F3759DF) - (i >> 1), _F32)
    for _ in range(3):
        y = y * (1.5 - 0.5 * x * y * y)
    return y


def _zero_vmem_2d(buf, rows):
    zv = jnp.zeros((16,), _F32)

    @pl.loop(0, rows)
    def _(i):
        for k in range(8):
            buf[i, pl.ds(k * 16, 16)] = zv


def _zero_vmem_1d(buf, n):
    zv = jnp.zeros((16,), _F32)

    @pl.loop(0, n // 16)
    def _(i):
        buf[pl.ds(i * 16, 16)] = zv


def _hist_chunks(idx_b, n_chunks, hist_sp, ones_b, sem):
    assert n_chunks % 16 == 0

    @pl.loop(0, n_chunks // 16)
    def _(i):
        descs = [
            pltpu.async_copy(ones_b, hist_sp.at[idx_b.at[i * 16 + k]], sem,
                             add=True)
            for k in range(16)
        ]
        for d in descs:
            d.wait()


def _edge_pipeline4(n_chunks, src, esrc_b, edst_b, agg_sp,
                    gbs, gsems, ssems, extra=None):
    assert n_chunks % 4 == 0
    nq = n_chunks // 4
    for k in range(4):
        pltpu.async_copy(src.at[esrc_b.at[k]], gbs[k], gsems[k])

    @pl.loop(0, nq)
    def _(i):
        base = i * 4
        ds = []
        for k in range(4):
            pltpu.make_async_copy(src.at[esrc_b.at[base + k]],
                                  gbs[k], gsems[k]).wait()
            ds.append(pltpu.async_copy(gbs[k], agg_sp.at[edst_b.at[base + k]],
                                       ssems[k], add=True))
        if extra is not None:
            extra(i)
        for k in range(4):
            ds[k].wait()

            @pl.when(base + k + 4 < n_chunks)
            def _(_k=k):
                pltpu.async_copy(src.at[esrc_b.at[base + 4 + _k]],
                                 gbs[_k], gsems[_k])


def _sc1_body(emb2, inp, esrc0, edst0, esrc1,
              agg_out, din0_out, dout1_out, h0_out,
              esrc_b, edst_b, esrc1_b, inp_b,
              gb0, gb1, gb2, gb3, scale_b, ones_b,
              agg_sp, hsrc_sp, hdst_sp, hout1_sp,
              gsem0, gsem1, gsem2, gsem3,
              ssem0, ssem1, ssem2, ssem3, wsem, hsem, stsem):
    c = lax.axis_index("c")
    s = lax.axis_index("s")
    gbs = (gb0, gb1, gb2, gb3)
    gsems = (gsem0, gsem1, gsem2, gsem3)
    ssems = (ssem0, ssem1, ssem2, ssem3)

    pltpu.sync_copy(esrc0.at[s], esrc_b)
    d_ed = pltpu.async_copy(edst0.at[s], edst_b, stsem)
    d_e1 = pltpu.async_copy(esrc1.at[s], esrc1_b, stsem)
    d_in = pltpu.async_copy(inp.at[s], inp_b, stsem)

    _zero_vmem_2d(gb0, 128)
    _zero_vmem_1d(scale_b, 640)

    @pl.loop(0, 8)
    def _(i):
        ones_b[pl.ds(i * 16, 16)] = jnp.ones((16,), _F32)

    pltpu.sync_copy(gb0, agg_sp.at[pl.ds(s * 256, 128)])
    pltpu.sync_copy(gb0, agg_sp.at[pl.ds(s * 256 + 128, 128)])
    pltpu.sync_copy(scale_b, hsrc_sp.at[pl.ds(s * 640, 640)])
    pltpu.sync_copy(scale_b.at[pl.ds(0, 256)], hdst_sp.at[pl.ds(s * 256, 256)])
    pltpu.sync_copy(scale_b.at[pl.ds(0, 256)], hout1_sp.at[pl.ds(s * 256, 256)])

    d_ed.wait()
    d_e1.wait()
    d_in.wait()

    @pl.loop(0, 5)
    def _(r):
        for k in range(8):
            v = inp_b[r, pl.ds(k * 16, 16)]
            inp_b[r, pl.ds(k * 16, 16)] = v * 2 + c

    plsc.subcore_barrier()

    for r in range(4):
        pltpu.async_copy(emb2.at[inp_b.at[r]], gbs[r], gsems[r])

    _hist_chunks(esrc_b, _C0, hsrc_sp, ones_b, hsem)
    plsc.subcore_barrier()

    pltpu.sync_copy(hsrc_sp.at[pl.ds(s * 640, 640)], scale_b)

    @pl.loop(0, 40)
    def _(i):
        d = scale_b[pl.ds(i * 16, 16)]
        scale_b[pl.ds(i * 16, 16)] = _rsqrt16(jnp.maximum(d, 1.0))

    wdescs = []
    for r in range(5):
        b, g = gbs[r % 4], gsems[r % 4]
        pltpu.make_async_copy(emb2.at[inp_b.at[r]], b, g).wait()

        @pl.loop(0, 128)
        def _(i, _r=r, _b=b):
            sc = plsc.load_gather(
                scale_b, [jnp.full((16,), _r * 128 + i, _I32)])
            for k in range(8):
                _b[i, pl.ds(k * 16, 16)] = _b[i, pl.ds(k * 16, 16)] * sc

        wdescs.append(pltpu.async_copy(
            b, h0_out.at[pl.ds(c * _N0P + s * 640 + r * 128, 128)], wsem))
        if r == 0:
            wdescs[0].wait()
            pltpu.async_copy(emb2.at[inp_b.at[4]], gb0, gsem0)

    @pl.loop(0, _C0)
    def _(j):
        for k in range(8):
            v = esrc_b[j, pl.ds(k * 16, 16)]
            esrc_b[j, pl.ds(k * 16, 16)] = v + c * _N0P

    for d in wdescs[1:]:
        d.wait()
    plsc.subcore_barrier()

    def _extra(i):
        dh = [
            pltpu.async_copy(ones_b, hdst_sp.at[edst_b.at[i * 4 + k]], hsem,
                             add=True)
            for k in range(4)
        ]

        @pl.when(i < _C1 // 4)
        def _():
            d1 = [
                pltpu.async_copy(ones_b, hout1_sp.at[esrc1_b.at[i * 4 + k]],
                                 hsem, add=True)
                for k in range(4)
            ]
            for d in d1:
                d.wait()

        for d in dh:
            d.wait()

    _edge_pipeline4(_C0, h0_out, esrc_b, edst_b, agg_sp,
                    gbs, gsems, ssems, extra=_extra)
    plsc.subcore_barrier()

    pltpu.sync_copy(agg_sp.at[pl.ds(s * 256, 256)],
                    agg_out.at[c, pl.ds(s * 256, 256)])

    @pl.when(jnp.logical_and(c == 0, s == 0))
    def _():
        pltpu.sync_copy(hdst_sp, din0_out)
        pltpu.sync_copy(hout1_sp, dout1_out)


def _sc2_body(h1p, esrc1, edst1,
              agg_out, din1_out,
              esrc_b, edst_b, gb0, gb1, gb2, gb3, zb, ones_b,
              h1_sp, agg_sp, hdst_sp,
              gsem0, gsem1, gsem2, gsem3,
              ssem0, ssem1, ssem2, ssem3, hsem):
    c = lax.axis_index("c")
    s = lax.axis_index("s")
    gbs = (gb0, gb1, gb2, gb3)
    gsems = (gsem0, gsem1, gsem2, gsem3)
    ssems = (ssem0, ssem1, ssem2, ssem3)

    pltpu.sync_copy(esrc1.at[s], esrc_b)
    pltpu.sync_copy(edst1.at[s], edst_b)

    d_h1 = pltpu.async_copy(h1p.at[c, pl.ds(s * 256, 256)],
                            h1_sp.at[pl.ds(s * 256, 256)], gsem0)

    _zero_vmem_2d(gb1, 64)
    _zero_vmem_1d(zb, 64)

    @pl.loop(0, 8)
    def _(i):
        ones_b[pl.ds(i * 16, 16)] = jnp.ones((16,), _F32)

    pltpu.sync_copy(gb1.at[pl.ds(0, 64)], agg_sp.at[pl.ds(s * 64, 64)])
    pltpu.sync_copy(zb, hdst_sp.at[pl.ds(s * 64, 64)])
    plsc.subcore_barrier()

    _hist_chunks(edst_b, _C1, hdst_sp, ones_b, hsem)
    d_h1.wait()
    plsc.subcore_barrier()

    _edge_pipeline4(_C1, h1_sp, esrc_b, edst_b, agg_sp,
                    gbs, gsems, ssems)
    plsc.subcore_barrier()

    pltpu.sync_copy(agg_sp.at[pl.ds(s * 64, 64)],
                    agg_out.at[c, pl.ds(s * 64, 64)])

    @pl.when(jnp.logical_and(c == 0, s == 0))
    def _():
        pltpu.sync_copy(hdst_sp, din1_out)


_MESH = plsc.VectorSubcoreMesh(core_axis_name="c", subcore_axis_name="s",
                               num_cores=_NC, num_subcores=_NS)
_SC_PARAMS = pltpu.CompilerParams(needs_layout_passes=False)

_sc1 = pl.kernel(
    _sc1_body,
    out_type=[
        jax.ShapeDtypeStruct((_NC, _N1P, _DH), _F32),
        jax.ShapeDtypeStruct((_N1P,), _F32),
        jax.ShapeDtypeStruct((_N1P,), _F32),
        jax.ShapeDtypeStruct((_NC * _N0P, _DH), _F32),
    ],
    mesh=_MESH,
    compiler_params=_SC_PARAMS,
    scratch_types=[
        pltpu.VMEM((_C0, 128), _I32),
        pltpu.VMEM((_C0, 128), _I32),
        pltpu.VMEM((_C1, 128), _I32),
        pltpu.VMEM((5, 128), _I32),
        pltpu.VMEM((128, 128), _F32),
        pltpu.VMEM((128, 128), _F32),
        pltpu.VMEM((128, 128), _F32),
        pltpu.VMEM((128, 128), _F32),
        pltpu.VMEM((640,), _F32),
        pltpu.VMEM((128,), _F32),
        pltpu.VMEM_SHARED((_N1P, _DH), _F32),
        pltpu.VMEM_SHARED((_N0P,), _F32),
        pltpu.VMEM_SHARED((_N1P,), _F32),
        pltpu.VMEM_SHARED((_N1P,), _F32),
        pltpu.SemaphoreType.DMA,
        pltpu.SemaphoreType.DMA,
        pltpu.SemaphoreType.DMA,
        pltpu.SemaphoreType.DMA,
        pltpu.SemaphoreType.DMA,
        pltpu.SemaphoreType.DMA,
        pltpu.SemaphoreType.DMA,
        pltpu.SemaphoreType.DMA,
        pltpu.SemaphoreType.DMA,
        pltpu.SemaphoreType.DMA,
        pltpu.SemaphoreType.DMA,
    ],
)

_sc2 = pl.kernel(
    _sc2_body,
    out_type=[
        jax.ShapeDtypeStruct((_NC, _N2P, _DH), _F32),
        jax.ShapeDtypeStruct((_N2P,), _F32),
    ],
    mesh=_MESH,
    compiler_params=_SC_PARAMS,
    scratch_types=[
        pltpu.VMEM((_C1, 128), _I32),
        pltpu.VMEM((_C1, 128), _I32),
        pltpu.VMEM((128, 128), _F32),
        pltpu.VMEM((128, 128), _F32),
        pltpu.VMEM((128, 128), _F32),
        pltpu.VMEM((128, 128), _F32),
        pltpu.VMEM((64,), _F32),
        pltpu.VMEM((128,), _F32),
        pltpu.VMEM_SHARED((_N1P, _DH), _F32),
        pltpu.VMEM_SHARED((_N2P, _DH), _F32),
        pltpu.VMEM_SHARED((_N2P,), _F32),
        pltpu.SemaphoreType.DMA,
        pltpu.SemaphoreType.DMA,
        pltpu.SemaphoreType.DMA,
        pltpu.SemaphoreType.DMA,
        pltpu.SemaphoreType.DMA,
        pltpu.SemaphoreType.DMA,
        pltpu.SemaphoreType.DMA,
        pltpu.SemaphoreType.DMA,
        pltpu.SemaphoreType.DMA,
    ],
)


def _tc1_body(agg_ref, din_ref, dout_ref, w_ref, b_ref, out_ref):
    x = jnp.concatenate([agg_ref[0], agg_ref[1]], axis=-1)
    x = x * lax.rsqrt(jnp.maximum(din_ref[...], 1.0))
    y = jnp.dot(x, w_ref[...], preferred_element_type=_F32) + b_ref[...]
    y = jnp.maximum(y, 0.0) * lax.rsqrt(jnp.maximum(dout_ref[...], 1.0))
    out_ref[0] = y[:, :_DH]
    out_ref[1] = y[:, _DH:]


def _tc2_body(agg_ref, din_ref, w1_ref, b1_ref, wo_ref, bo_ref, out_ref):
    x = jnp.concatenate([agg_ref[0], agg_ref[1]], axis=-1)
    x = x * lax.rsqrt(jnp.maximum(din_ref[...], 1.0))
    h = jnp.dot(x, w1_ref[...], preferred_element_type=_F32) + b1_ref[...]
    y = jnp.dot(h, wo_ref[...], preferred_element_type=_F32) + bo_ref[...]
    out_ref[...] = y[:_N2]


_tc1 = pl.pallas_call(
    _tc1_body,
    out_shape=jax.ShapeDtypeStruct((_NC, _N1P, _DH), _F32),
)

_tc2 = pl.pallas_call(
    _tc2_body,
    out_shape=jax.ShapeDtypeStruct((_N2, _OUT), _F32),
)


def kernel(input_nodes, edge_src0, edge_dst0, edge_src1, edge_dst1,
           emb, W0, b0, W1, b1, Wout, bout):
    emb2 = emb.reshape(_NUM_NODES * _NC, _DH)
    inp = jnp.concatenate(
        [input_nodes.astype(_I32), jnp.zeros((_N0P - _N0,), _I32)]
    ).reshape(_NS, 5, 128)
    es0 = jnp.concatenate(
        [edge_src0.astype(_I32), jnp.full((_E0P - _E0,), _N0, _I32)]
    ).reshape(_NS, _C0, 128)
    ed0 = jnp.concatenate(
        [edge_dst0.astype(_I32), jnp.full((_E0P - _E0,), _N1, _I32)]
    ).reshape(_NS, _C0, 128)
    es1 = jnp.concatenate(
        [edge_src1.astype(_I32), jnp.full((_E1P - _E1,), _N1, _I32)]
    ).reshape(_NS, _C1, 128)
    ed1 = jnp.concatenate(
        [edge_dst1.astype(_I32), jnp.full((_E1P - _E1,), _N2, _I32)]
    ).reshape(_NS, _C1, 128)

    agg0, din0, dout1, _ = _sc1(emb2, inp, es0, ed0, es1)
    h1 = _tc1(agg0, din0.reshape(_N1P, 1), dout1.reshape(_N1P, 1),
              W0, b0.reshape(1, _HID))
    agg1, din1 = _sc2(h1, es1, ed1)
    logits = _tc2(agg1, din1.reshape(_N2P, 1),
                  W1, b1.reshape(1, _HID), Wout, bout.reshape(1, _OUT))
    return logits
